# Initial kernel scaffold; baseline (speedup 1.0000x reference)
import jax
import jax.numpy as jnp
from jax.experimental import pallas as pl


def kernel(x, batch, W1, b1, W2):
    raise NotImplementedError("write your pallas kernel here")


# speedup vs baseline: 14.0052x; 14.0052x over previous
import jax
import jax.numpy as jnp
from jax.experimental import pallas as pl
from jax.experimental.pallas import tpu as pltpu

G = 256
D = 128
H = 32


def _pool_body(nblocks, blk, npad):
    BIG = npad + 1

    def body(batch_ref, x_ref, W1_ref, b1_ref, W2_ref, out_ref,
             acc_ref, m_ref, s_ref, rbc_ref, prev_ref):
        k = pl.program_id(0)

        @pl.when(k == 0)
        def _init():
            acc_ref[...] = jnp.zeros_like(acc_ref)
            m_ref[...] = jnp.full_like(m_ref, -1e30)
            s_ref[...] = jnp.zeros_like(s_ref)
            rbc_ref[...] = jnp.zeros_like(rbc_ref)
            prev_ref[0] = jnp.int32(-1)

        b = batch_ref[0, 0, :]
        xb = x_ref[...]
        W1a = W1_ref[0:D, :]
        W1b = W1_ref[D:2 * D, :]

        gid = jax.lax.broadcasted_iota(jnp.int32, (blk, G), 1)
        iotaB = jax.lax.broadcasted_iota(jnp.int32, (blk, G), 0)
        Ob = b[:, None] == gid
        Of = Ob.astype(jnp.float32)

        xwa = jnp.dot(xb, W1a, preferred_element_type=jnp.float32)
        xwb = jnp.dot(xb, W1b, preferred_element_type=jnp.float32)

        posg = jnp.min(jnp.where(Ob, iotaB, BIG), axis=0, keepdims=True)
        R = (iotaB == posg).astype(jnp.float32)
        candT = jax.lax.dot_general(
            xwb, R, (((0,), (0,)), ((), ())),
            preferred_element_type=jnp.float32)
        prev = prev_ref[0]
        gvec = jax.lax.broadcasted_iota(jnp.int32, (1, G), 1)
        newm = (posg < BIG) & (gvec > prev)
        rbc_ref[...] = jnp.where(newm, candT, rbc_ref[...])
        prev_ref[0] = batch_ref[0, 0, blk - 1]

        rbc_n = jax.lax.dot_general(
            Of, rbc_ref[...], (((1,), (1,)), ((), ())),
            preferred_element_type=jnp.float32)
        h = jnp.tanh(xwa + rbc_n + b1_ref[...])
        beta = jnp.dot(h, W2_ref[...],
                       preferred_element_type=jnp.float32)

        mb = jnp.max(jnp.where(Ob, beta, -1e30), axis=0, keepdims=True)
        m_old = m_ref[...]
        m_new = jnp.maximum(m_old, mb)
        scale = jnp.exp(m_old - m_new)
        m_ref[...] = m_new
        mnode = jax.lax.dot_general(
            Of, m_new, (((1,), (1,)), ((), ())),
            preferred_element_type=jnp.float32)
        e = jnp.exp(beta - mnode)
        eOf = Of * e
        s_ref[...] = s_ref[...] * scale + jnp.sum(eOf, axis=0, keepdims=True)
        acc_ref[...] = acc_ref[...] * scale + jax.lax.dot_general(
            xb, eOf, (((0,), (0,)), ((), ())),
            preferred_element_type=jnp.float32)

        @pl.when(k == nblocks - 1)
        def _finish():
            out_ref[...] = (acc_ref[...] / (s_ref[...] + 1e-16)).T

    return body


def kernel(x, batch, W1, b1, W2):
    n = x.shape[0]
    blk = 2048
    nblocks = (n + blk - 1) // blk
    npad = nblocks * blk
    if npad != n:
        x = jnp.concatenate(
            [x, jnp.zeros((npad - n, D), jnp.float32)], axis=0)
        batch = jnp.concatenate(
            [batch, jnp.full((npad - n,), G, jnp.int32)], axis=0)
    batch3 = batch.reshape(nblocks, 1, blk)
    b1r = b1.reshape(1, H)

    out = pl.pallas_call(
        _pool_body(nblocks, blk, npad),
        grid=(nblocks,),
        in_specs=[
            pl.BlockSpec((1, 1, blk), lambda k: (k, 0, 0)),
            pl.BlockSpec((blk, D), lambda k: (k, 0)),
            pl.BlockSpec((2 * D, H), lambda k: (0, 0)),
            pl.BlockSpec((1, H), lambda k: (0, 0)),
            pl.BlockSpec((H, 1), lambda k: (0, 0)),
        ],
        out_specs=pl.BlockSpec((G, D), lambda k: (0, 0)),
        out_shape=jax.ShapeDtypeStruct((G, D), jnp.float32),
        scratch_shapes=[
            pltpu.VMEM((D, G), jnp.float32),
            pltpu.VMEM((1, G), jnp.float32),
            pltpu.VMEM((1, G), jnp.float32),
            pltpu.VMEM((H, G), jnp.float32),
            pltpu.SMEM((1,), jnp.int32),
        ],
        compiler_params=pltpu.CompilerParams(
            dimension_semantics=("arbitrary",)),
    )(batch3, x, W1, b1r, W2)
    return out


# speedup vs baseline: 15.1792x; 1.0838x over previous
import functools

import jax
import jax.numpy as jnp
from jax import lax
from jax.experimental import pallas as pl
from jax.experimental.pallas import tpu as pltpu
from jax.experimental.pallas import tpu_sc as plsc

G = 256
D = 128
H = 32


def _roots_sc(x, batch_pad, n):
    NP = batch_pad.shape[0]
    NW = 16
    C = NP // NW
    T = 512
    mesh = plsc.VectorSubcoreMesh(core_axis_name="c", subcore_axis_name="s")

    @functools.partial(
        pl.kernel, mesh=mesh,
        out_type=jax.ShapeDtypeStruct((G, D), jnp.float32),
        scratch_types=[
            pltpu.VMEM((C + 16,), jnp.int32),
            pltpu.VMEM((T,), jnp.int32),
            pltpu.VMEM_SHARED((NW, T), jnp.int32),
            pltpu.VMEM((NW, T), jnp.int32),
            pltpu.VMEM((16, D), jnp.float32),
            pltpu.SemaphoreType.DMA,
        ],
        compiler_params=pltpu.CompilerParams(needs_layout_passes=False),
    )
    def k(batch_hbm, x_hbm, roots_hbm, buf, tbl, shared, tball, rows, sem):
        cid = lax.axis_index("c")
        sid = lax.axis_index("s")

        @pl.when(cid == 0)
        def _work():
            lo = sid * C
            zz = jnp.zeros((16,), jnp.int32)
            for t0 in range(0, T, 16):
                tbl[pl.ds(t0, 16)] = zz
            pltpu.sync_copy(batch_hbm.at[pl.ds(lo, C)], buf.at[pl.ds(16, C)])

            @pl.when(sid == 0)
            def _first():
                buf[pl.ds(0, 16)] = jnp.full((16,), -1, jnp.int32)

            @pl.when(sid > 0)
            def _rest():
                pltpu.sync_copy(batch_hbm.at[pl.ds(lo - 16, 16)],
                                buf.at[pl.ds(0, 16)])

            iota = lax.iota(jnp.int32, 16)

            def step(j, carry):
                cur = buf[pl.ds(16 + 16 * j, 16)]
                prv = buf[pl.ds(15 + 16 * j, 16)]
                bmask = cur != prv
                vals = lo + 16 * j + iota + 1
                plsc.store_scatter(tbl, [cur], vals, mask=bmask)
                return carry

            lax.fori_loop(0, C // 16, step, 0)

            pltpu.sync_copy(tbl, shared.at[sid])
            plsc.subcore_barrier()

            pltpu.sync_copy(shared, tball)
            acc = jnp.zeros((16,), jnp.int32)
            for t in range(NW):
                acc = acc + tball[t, pl.ds(16 * sid, 16)]
            fi = jnp.clip(acc - 1, 0, n - 1)
            pltpu.async_copy(x_hbm.at[fi], rows, sem).wait()
            pltpu.sync_copy(rows, roots_hbm.at[pl.ds(16 * sid, 16)])

    return k(batch_pad, x)


def _pool_body(nblocks, blk):
    def body(batch_ref, x_ref, roots_ref, W1_ref, b1_ref, W2_ref, out_ref,
             acc_ref, m_ref, s_ref, rbc_ref):
        k = pl.program_id(0)
        W1b = W1_ref[D:2 * D, :]

        @pl.when(k == 0)
        def _init():
            acc_ref[...] = jnp.zeros_like(acc_ref)
            m_ref[...] = jnp.full_like(m_ref, -1e30)
            s_ref[...] = jnp.zeros_like(s_ref)
            rbc_ref[...] = jax.lax.dot_general(
                W1b, roots_ref[...], (((0,), (1,)), ((), ())),
                preferred_element_type=jnp.float32)

        b = batch_ref[0, 0, :]
        xb = x_ref[...]
        W1a = W1_ref[0:D, :]

        gid = jax.lax.broadcasted_iota(jnp.int32, (blk, G), 1)
        Ob = b[:, None] == gid
        Of = Ob.astype(jnp.float32)

        xwa = jnp.dot(xb, W1a, preferred_element_type=jnp.float32)

        rbc_n = jax.lax.dot_general(
            Of, rbc_ref[...], (((1,), (1,)), ((), ())),
            preferred_element_type=jnp.float32)
        h = jnp.tanh(xwa + rbc_n + b1_ref[...])
        beta = jnp.dot(h, W2_ref[...],
                       preferred_element_type=jnp.float32)

        mb = jnp.max(jnp.where(Ob, beta, -1e30), axis=0, keepdims=True)
        m_old = m_ref[...]
        m_new = jnp.maximum(m_old, mb)
        scale = jnp.exp(m_old - m_new)
        m_ref[...] = m_new
        mnode = jax.lax.dot_general(
            Of, m_new, (((1,), (1,)), ((), ())),
            preferred_element_type=jnp.float32)
        e = jnp.exp(beta - mnode)
        eOf = Of * e
        s_ref[...] = s_ref[...] * scale + jnp.sum(eOf, axis=0, keepdims=True)
        acc_ref[...] = acc_ref[...] * scale + jax.lax.dot_general(
            xb, eOf, (((0,), (0,)), ((), ())),
            preferred_element_type=jnp.float32)

        @pl.when(k == nblocks - 1)
        def _finish():
            out_ref[...] = (acc_ref[...] / (s_ref[...] + 1e-16)).T

    return body


def _pool_tc(x_pad, batch3, roots, W1, b1r, W2, nblocks, blk):
    return pl.pallas_call(
        _pool_body(nblocks, blk),
        grid=(nblocks,),
        in_specs=[
            pl.BlockSpec((1, 1, blk), lambda k: (k, 0, 0)),
            pl.BlockSpec((blk, D), lambda k: (k, 0)),
            pl.BlockSpec((G, D), lambda k: (0, 0)),
            pl.BlockSpec((2 * D, H), lambda k: (0, 0)),
            pl.BlockSpec((1, H), lambda k: (0, 0)),
            pl.BlockSpec((H, 1), lambda k: (0, 0)),
        ],
        out_specs=pl.BlockSpec((G, D), lambda k: (0, 0)),
        out_shape=jax.ShapeDtypeStruct((G, D), jnp.float32),
        scratch_shapes=[
            pltpu.VMEM((D, G), jnp.float32),
            pltpu.VMEM((1, G), jnp.float32),
            pltpu.VMEM((1, G), jnp.float32),
            pltpu.VMEM((H, G), jnp.float32),
        ],
        compiler_params=pltpu.CompilerParams(
            dimension_semantics=("arbitrary",)),
    )(batch3, x_pad, roots, W1, b1r, W2)


def kernel(x, batch, W1, b1, W2):
    n = x.shape[0]
    blk = 2048
    nblocks = (n + blk - 1) // blk
    npad = nblocks * blk
    if npad != n:
        x_pad = jnp.concatenate(
            [x, jnp.zeros((npad - n, D), jnp.float32)], axis=0)
        batch_pad = jnp.concatenate(
            [batch, jnp.full((npad - n,), G, jnp.int32)], axis=0)
    else:
        x_pad, batch_pad = x, batch
    batch3 = batch_pad.reshape(nblocks, 1, blk)
    b1r = b1.reshape(1, H)

    roots = _roots_sc(x, batch_pad, n)
    return _pool_tc(x_pad, batch3, roots, W1, b1r, W2, nblocks, blk)


# speedup vs baseline: 19.4344x; 1.2803x over previous
import functools

import jax
import jax.numpy as jnp
from jax import lax
from jax.experimental import pallas as pl
from jax.experimental.pallas import tpu as pltpu
from jax.experimental.pallas import tpu_sc as plsc

G = 256
D = 128
H = 32


def _roots_sc(x, batch_pad, n):
    NP = batch_pad.shape[0]
    NW = 16
    C = NP // NW
    T = 512
    mesh = plsc.VectorSubcoreMesh(core_axis_name="c", subcore_axis_name="s")

    @functools.partial(
        pl.kernel, mesh=mesh,
        out_type=jax.ShapeDtypeStruct((G, D), jnp.float32),
        scratch_types=[
            pltpu.VMEM((C + 16,), jnp.int32),
            pltpu.VMEM((T,), jnp.int32),
            pltpu.VMEM_SHARED((NW, T), jnp.int32),
            pltpu.VMEM((NW, T), jnp.int32),
            pltpu.VMEM((16, D), jnp.float32),
            pltpu.SemaphoreType.DMA,
        ],
        compiler_params=pltpu.CompilerParams(needs_layout_passes=False),
    )
    def k(batch_hbm, x_hbm, roots_hbm, buf, tbl, shared, tball, rows, sem):
        cid = lax.axis_index("c")
        sid = lax.axis_index("s")

        @pl.when(cid == 0)
        def _work():
            lo = sid * C
            zz = jnp.zeros((16,), jnp.int32)
            for t0 in range(0, T, 16):
                tbl[pl.ds(t0, 16)] = zz
            pltpu.sync_copy(batch_hbm.at[pl.ds(lo, C)], buf.at[pl.ds(16, C)])

            @pl.when(sid == 0)
            def _first():
                buf[pl.ds(0, 16)] = jnp.full((16,), -1, jnp.int32)

            @pl.when(sid > 0)
            def _rest():
                pltpu.sync_copy(batch_hbm.at[pl.ds(lo - 16, 16)],
                                buf.at[pl.ds(0, 16)])

            iota = lax.iota(jnp.int32, 16)

            def step(j, carry):
                cur = buf[pl.ds(16 + 16 * j, 16)]
                prv = buf[pl.ds(15 + 16 * j, 16)]
                bmask = cur != prv
                vals = lo + 16 * j + iota + 1
                plsc.store_scatter(tbl, [cur], vals, mask=bmask)
                return carry

            lax.fori_loop(0, C // 16, step, 0)

            pltpu.sync_copy(tbl, shared.at[sid])
            plsc.subcore_barrier()

            pltpu.sync_copy(shared, tball)
            acc = jnp.zeros((16,), jnp.int32)
            for t in range(NW):
                acc = acc + tball[t, pl.ds(16 * sid, 16)]
            fi = jnp.clip(acc - 1, 0, n - 1)
            pltpu.async_copy(x_hbm.at[fi], rows, sem).wait()
            pltpu.sync_copy(rows, roots_hbm.at[pl.ds(16 * sid, 16)])

    return k(batch_pad, x)


def _pool_body(nblocks, blk):
    def body(batch_ref, x_ref, roots_ref, W1_ref, b1_ref, W2_ref, out_ref,
             acc_ref, s_ref, rbc_ref):
        k = pl.program_id(0)
        bf = jnp.bfloat16

        @pl.when(k == 0)
        def _init():
            acc_ref[...] = jnp.zeros_like(acc_ref)
            s_ref[...] = jnp.zeros_like(s_ref)
            roots = roots_ref[...]
            W1a0 = W1_ref[0:D, :]
            W1b0 = W1_ref[D:2 * D, :]
            rbc = jnp.dot(roots, W1b0,
                          preferred_element_type=jnp.float32)
            pre_r = jnp.dot(roots, W1a0,
                            preferred_element_type=jnp.float32) \
                + rbc + b1_ref[...]
            beta_r = jnp.dot(jnp.tanh(pre_r), W2_ref[...],
                             preferred_element_type=jnp.float32)
            rbc_ref[:, 0:H] = rbc.astype(bf)
            rbc_ref[:, H:H + 1] = beta_r.astype(bf)

        b = batch_ref[0, 0, :]
        xb = x_ref[...]
        W1a = W1_ref[0:D, :]

        gid = jax.lax.broadcasted_iota(jnp.int32, (1, G), 1).astype(bf)
        b_bf = b.astype(bf)
        Ob = b_bf[:, None] == gid
        Of = Ob.astype(bf)

        xwa = jnp.dot(xb, W1a, preferred_element_type=jnp.float32)

        gath = jnp.dot(Of, rbc_ref[...],
                       preferred_element_type=jnp.float32)
        rbc_n = gath[:, 0:H]
        mnode = gath[:, H:H + 1]
        h = jnp.tanh(xwa + rbc_n + b1_ref[...])
        beta = jnp.dot(h, W2_ref[...],
                       preferred_element_type=jnp.float32)

        e = jnp.exp(beta - mnode)
        eOf = Of * e.astype(bf)
        s_ref[...] = s_ref[...] + jax.lax.dot_general(
            jnp.ones((1, blk), bf), eOf, (((1,), (0,)), ((), ())),
            preferred_element_type=jnp.float32)
        acc_ref[...] = acc_ref[...] + jax.lax.dot_general(
            xb.astype(bf), eOf, (((0,), (0,)), ((), ())),
            preferred_element_type=jnp.float32)

        @pl.when(k == nblocks - 1)
        def _finish():
            out_ref[...] = (acc_ref[...] / (s_ref[...] + 1e-16)).T

    return body


def _pool_tc(x_pad, batch3, roots, W1, b1r, W2, nblocks, blk):
    return pl.pallas_call(
        _pool_body(nblocks, blk),
        grid=(nblocks,),
        in_specs=[
            pl.BlockSpec((1, 1, blk), lambda k: (k, 0, 0)),
            pl.BlockSpec((blk, D), lambda k: (k, 0)),
            pl.BlockSpec((G, D), lambda k: (0, 0)),
            pl.BlockSpec((2 * D, H), lambda k: (0, 0)),
            pl.BlockSpec((1, H), lambda k: (0, 0)),
            pl.BlockSpec((H, 1), lambda k: (0, 0)),
        ],
        out_specs=pl.BlockSpec((G, D), lambda k: (0, 0)),
        out_shape=jax.ShapeDtypeStruct((G, D), jnp.float32),
        scratch_shapes=[
            pltpu.VMEM((D, G), jnp.float32),
            pltpu.VMEM((1, G), jnp.float32),
            pltpu.VMEM((G, H + 1), jnp.bfloat16),
        ],
        compiler_params=pltpu.CompilerParams(
            dimension_semantics=("arbitrary",)),
    )(batch3, x_pad, roots, W1, b1r, W2)


def kernel(x, batch, W1, b1, W2):
    n = x.shape[0]
    blk = 2048
    nblocks = (n + blk - 1) // blk
    npad = nblocks * blk
    if npad != n:
        x_pad = jnp.concatenate(
            [x, jnp.zeros((npad - n, D), jnp.float32)], axis=0)
        batch_pad = jnp.concatenate(
            [batch, jnp.full((npad - n,), G, jnp.int32)], axis=0)
    else:
        x_pad, batch_pad = x, batch
    batch3 = batch_pad.reshape(nblocks, 1, blk)
    b1r = b1.reshape(1, H)

    roots = _roots_sc(x, batch_pad, n)
    return _pool_tc(x_pad, batch3, roots, W1, b1r, W2, nblocks, blk)


# speedup vs baseline: 26.4204x; 1.3595x over previous
import functools

import jax
import jax.numpy as jnp
from jax import lax
from jax.experimental import pallas as pl
from jax.experimental.pallas import tpu as pltpu
from jax.experimental.pallas import tpu_sc as plsc

G = 256
D = 128
H = 32


def _roots_sc(x, batch_pad, n):
    NP = batch_pad.shape[0]
    NW = 16
    C = NP // NW
    T = 512
    mesh = plsc.VectorSubcoreMesh(core_axis_name="c", subcore_axis_name="s")

    @functools.partial(
        pl.kernel, mesh=mesh,
        out_type=jax.ShapeDtypeStruct((G, D), jnp.float32),
        scratch_types=[
            pltpu.VMEM((C + 16,), jnp.int32),
            pltpu.VMEM((T,), jnp.int32),
            pltpu.VMEM_SHARED((NW, T), jnp.int32),
            pltpu.VMEM((NW, T), jnp.int32),
            pltpu.VMEM((16, D), jnp.float32),
            pltpu.SemaphoreType.DMA,
        ],
        compiler_params=pltpu.CompilerParams(needs_layout_passes=False),
    )
    def k(batch_hbm, x_hbm, roots_hbm, buf, tbl, shared, tball, rows, sem):
        cid = lax.axis_index("c")
        sid = lax.axis_index("s")

        @pl.when(cid == 0)
        def _work():
            lo = sid * C
            zz = jnp.zeros((16,), jnp.int32)
            for t0 in range(0, T, 16):
                tbl[pl.ds(t0, 16)] = zz
            pltpu.sync_copy(batch_hbm.at[pl.ds(lo, C)], buf.at[pl.ds(16, C)])

            @pl.when(sid == 0)
            def _first():
                buf[pl.ds(0, 16)] = jnp.full((16,), -1, jnp.int32)

            @pl.when(sid > 0)
            def _rest():
                pltpu.sync_copy(batch_hbm.at[pl.ds(lo - 16, 16)],
                                buf.at[pl.ds(0, 16)])

            iota = lax.iota(jnp.int32, 16)

            def step(j, carry):
                cur = buf[pl.ds(16 + 16 * j, 16)]
                prv = buf[pl.ds(15 + 16 * j, 16)]
                bmask = cur != prv
                vals = lo + 16 * j + iota + 1
                plsc.store_scatter(tbl, [cur], vals, mask=bmask)
                return carry

            lax.fori_loop(0, C // 16, step, 0)

            pltpu.sync_copy(tbl, shared.at[sid])
            plsc.subcore_barrier()

            pltpu.sync_copy(shared, tball)
            acc = jnp.zeros((16,), jnp.int32)
            for t in range(NW):
                acc = acc + tball[t, pl.ds(16 * sid, 16)]
            fi = jnp.clip(acc - 1, 0, n - 1)
            pltpu.async_copy(x_hbm.at[fi], rows, sem).wait()
            pltpu.sync_copy(rows, roots_hbm.at[pl.ds(16 * sid, 16)])

    return k(batch_pad, x)


def _pool_body(nblocks, blk):
    def body(batch_ref, x_ref, roots_ref, W1_ref, b1_ref, W2_ref, out_ref,
             acc_ref, s_ref, rbc_ref):
        k = pl.program_id(0)
        bf = jnp.bfloat16

        @pl.when(k == 0)
        def _init():
            acc_ref[...] = jnp.zeros_like(acc_ref)
            s_ref[...] = jnp.zeros_like(s_ref)
            roots = roots_ref[...]
            W1a0 = W1_ref[0:D, :]
            W1b0 = W1_ref[D:2 * D, :]
            rbc = jnp.dot(roots, W1b0,
                          preferred_element_type=jnp.float32)
            pre_r = jnp.dot(roots, W1a0,
                            preferred_element_type=jnp.float32) \
                + rbc + b1_ref[...]
            beta_r = jnp.dot(jnp.tanh(pre_r), W2_ref[...],
                             preferred_element_type=jnp.float32)
            rbc_ref[:, 0:H] = rbc.astype(bf)
            rbc_ref[:, H:H + 1] = beta_r.astype(bf)

        b = batch_ref[0, 0, :]
        xb = x_ref[...]
        W1a = W1_ref[0:D, :]

        gid = jax.lax.broadcasted_iota(jnp.int32, (1, G), 1).astype(bf)
        b_bf = b.astype(bf)
        d = b_bf[:, None] - gid
        Of = jnp.maximum(bf(1.0) - jnp.abs(d), bf(0.0))

        xwa = jnp.dot(xb, W1a, preferred_element_type=jnp.float32)

        gath = jnp.dot(Of, rbc_ref[...],
                       preferred_element_type=jnp.float32)
        rbc_n = gath[:, 0:H]
        mnode = gath[:, H:H + 1]
        h = jnp.tanh(xwa + rbc_n + b1_ref[...])
        beta = jnp.dot(h, W2_ref[...],
                       preferred_element_type=jnp.float32)

        e = jnp.exp(beta - mnode)
        eOf = Of * e.astype(bf)
        s_ref[...] = s_ref[...] + jax.lax.dot_general(
            jnp.ones((1, blk), bf), eOf, (((1,), (0,)), ((), ())),
            preferred_element_type=jnp.float32)
        acc_ref[...] = acc_ref[...] + jax.lax.dot_general(
            xb.astype(bf), eOf, (((0,), (0,)), ((), ())),
            preferred_element_type=jnp.float32)

        @pl.when(k == nblocks - 1)
        def _finish():
            out_ref[...] = (acc_ref[...] / (s_ref[...] + 1e-16)).T

    return body


def _pool_tc(x_pad, batch3, roots, W1, b1r, W2, nblocks, blk):
    return pl.pallas_call(
        _pool_body(nblocks, blk),
        grid=(nblocks,),
        in_specs=[
            pl.BlockSpec((1, 1, blk), lambda k: (k, 0, 0)),
            pl.BlockSpec((blk, D), lambda k: (k, 0)),
            pl.BlockSpec((G, D), lambda k: (0, 0)),
            pl.BlockSpec((2 * D, H), lambda k: (0, 0)),
            pl.BlockSpec((1, H), lambda k: (0, 0)),
            pl.BlockSpec((H, 1), lambda k: (0, 0)),
        ],
        out_specs=pl.BlockSpec((G, D), lambda k: (0, 0)),
        out_shape=jax.ShapeDtypeStruct((G, D), jnp.float32),
        scratch_shapes=[
            pltpu.VMEM((D, G), jnp.float32),
            pltpu.VMEM((1, G), jnp.float32),
            pltpu.VMEM((G, H + 1), jnp.bfloat16),
        ],
        compiler_params=pltpu.CompilerParams(
            dimension_semantics=("arbitrary",)),
    )(batch3, x_pad, roots, W1, b1r, W2)


def kernel(x, batch, W1, b1, W2):
    n = x.shape[0]
    blk = 2000
    assert n % blk == 0
    nblocks = n // blk
    batch3 = batch.reshape(nblocks, 1, blk)
    b1r = b1.reshape(1, H)

    sc_pad = (-n) % (16 * 16)
    batch_sc = jnp.concatenate(
        [batch, jnp.full((sc_pad,), G, jnp.int32)]) if sc_pad else batch

    roots = _roots_sc(x, batch_sc, n)
    return _pool_tc(x, batch3, roots, W1, b1r, W2, nblocks, blk)


# speedup vs baseline: 26.6649x; 1.0093x over previous
import functools

import jax
import jax.numpy as jnp
from jax import lax
from jax.experimental import pallas as pl
from jax.experimental.pallas import tpu as pltpu
from jax.experimental.pallas import tpu_sc as plsc

G = 256
D = 128
H = 32


def _roots_sc(x, batch_pad, n):
    NP = batch_pad.shape[0]
    NW = 16
    C = NP // NW
    T = 512
    mesh = plsc.VectorSubcoreMesh(core_axis_name="c", subcore_axis_name="s")

    @functools.partial(
        pl.kernel, mesh=mesh,
        out_type=jax.ShapeDtypeStruct((G, D), jnp.float32),
        scratch_types=[
            pltpu.VMEM((C + 16,), jnp.int32),
            pltpu.VMEM((T,), jnp.int32),
            pltpu.VMEM_SHARED((NW, T), jnp.int32),
            pltpu.VMEM((NW, T), jnp.int32),
            pltpu.VMEM((16, D), jnp.float32),
            pltpu.SemaphoreType.DMA,
        ],
        compiler_params=pltpu.CompilerParams(needs_layout_passes=False),
    )
    def k(batch_hbm, x_hbm, roots_hbm, buf, tbl, shared, tball, rows, sem):
        cid = lax.axis_index("c")
        sid = lax.axis_index("s")

        @pl.when(cid == 0)
        def _work():
            lo = sid * C
            zz = jnp.zeros((16,), jnp.int32)
            for t0 in range(0, T, 16):
                tbl[pl.ds(t0, 16)] = zz
            pltpu.sync_copy(batch_hbm.at[pl.ds(lo, C)], buf.at[pl.ds(16, C)])

            @pl.when(sid == 0)
            def _first():
                buf[pl.ds(0, 16)] = jnp.full((16,), -1, jnp.int32)

            @pl.when(sid > 0)
            def _rest():
                pltpu.sync_copy(batch_hbm.at[pl.ds(lo - 16, 16)],
                                buf.at[pl.ds(0, 16)])

            iota = lax.iota(jnp.int32, 16)

            def step(j, carry):
                cur = buf[pl.ds(16 + 16 * j, 16)]
                prv = buf[pl.ds(15 + 16 * j, 16)]
                bmask = cur != prv
                vals = lo + 16 * j + iota + 1
                plsc.store_scatter(tbl, [cur], vals, mask=bmask)
                return carry

            lax.fori_loop(0, C // 16, step, 0)

            pltpu.sync_copy(tbl, shared.at[sid])
            plsc.subcore_barrier()

            pltpu.sync_copy(shared, tball)
            acc = jnp.zeros((16,), jnp.int32)
            for t in range(NW):
                acc = acc + tball[t, pl.ds(16 * sid, 16)]
            fi = jnp.clip(acc - 1, 0, n - 1)
            pltpu.async_copy(x_hbm.at[fi], rows, sem).wait()
            pltpu.sync_copy(rows, roots_hbm.at[pl.ds(16 * sid, 16)])

    return k(batch_pad, x)


def _pool_body(nblocks, blk):
    def body(batch_ref, x_ref, roots_ref, W1_ref, b1_ref, W2_ref, out_ref,
             acc_ref, s_ref, rbc_ref):
        k = pl.program_id(0)
        bf = jnp.bfloat16

        @pl.when(k == 0)
        def _init():
            acc_ref[...] = jnp.zeros_like(acc_ref)
            s_ref[...] = jnp.zeros_like(s_ref)
            roots = roots_ref[...]
            W1a0 = W1_ref[0:D, :]
            W1b0 = W1_ref[D:2 * D, :]
            rbc = jnp.dot(roots, W1b0,
                          preferred_element_type=jnp.float32)
            pre_r = jnp.dot(roots, W1a0,
                            preferred_element_type=jnp.float32) \
                + rbc + b1_ref[...]
            beta_r = jnp.dot(jnp.tanh(pre_r), W2_ref[...],
                             preferred_element_type=jnp.float32)
            tab = jnp.concatenate([rbc, beta_r], axis=1)
            rbc_ref[...] = tab.astype(bf).T

        b = batch_ref[0, 0, :]
        xb = x_ref[...]
        W1a = W1_ref[0:D, :]

        gid = jax.lax.broadcasted_iota(jnp.int32, (G, 1), 0).astype(bf)
        b_bf = b[None, :].astype(bf)
        dd = gid - b_bf
        OfT = jnp.maximum(bf(1.0) - jnp.abs(dd), bf(0.0))

        xwaT = jnp.dot(xb, W1a,
                       preferred_element_type=jnp.float32).T

        gathT = jnp.dot(rbc_ref[...].astype(bf), OfT,
                        preferred_element_type=jnp.float32)
        hT = jnp.tanh(xwaT + gathT[0:H, :] + b1_ref[...].T)
        betaT = jax.lax.dot_general(
            W2_ref[...], hT, (((0,), (0,)), ((), ())),
            preferred_element_type=jnp.float32)

        eT = jnp.exp(betaT - gathT[H:H + 1, :])
        eOfT = OfT * eT.astype(bf)
        s_ref[...] = s_ref[...] + jnp.dot(
            eOfT, jnp.ones((blk, 1), bf),
            preferred_element_type=jnp.float32)
        acc_ref[...] = acc_ref[...] + jnp.dot(
            eOfT, xb.astype(bf),
            preferred_element_type=jnp.float32)

        @pl.when(k == nblocks - 1)
        def _finish():
            out_ref[...] = acc_ref[...] / (s_ref[...] + 1e-16)

    return body


def _pool_tc(x_pad, batch3, roots, W1, b1r, W2, nblocks, blk):
    return pl.pallas_call(
        _pool_body(nblocks, blk),
        grid=(nblocks,),
        in_specs=[
            pl.BlockSpec((1, 1, blk), lambda k: (k, 0, 0)),
            pl.BlockSpec((blk, D), lambda k: (k, 0)),
            pl.BlockSpec((G, D), lambda k: (0, 0)),
            pl.BlockSpec((2 * D, H), lambda k: (0, 0)),
            pl.BlockSpec((1, H), lambda k: (0, 0)),
            pl.BlockSpec((H, 1), lambda k: (0, 0)),
        ],
        out_specs=pl.BlockSpec((G, D), lambda k: (0, 0)),
        out_shape=jax.ShapeDtypeStruct((G, D), jnp.float32),
        scratch_shapes=[
            pltpu.VMEM((G, D), jnp.float32),
            pltpu.VMEM((G, 1), jnp.float32),
            pltpu.VMEM((H + 1, G), jnp.bfloat16),
        ],
        compiler_params=pltpu.CompilerParams(
            dimension_semantics=("arbitrary",)),
    )(batch3, x_pad, roots, W1, b1r, W2)


def kernel(x, batch, W1, b1, W2):
    n = x.shape[0]
    blk = 2000
    assert n % blk == 0
    nblocks = n // blk
    batch3 = batch.reshape(nblocks, 1, blk)
    b1r = b1.reshape(1, H)

    sc_pad = (-n) % (16 * 16)
    batch_sc = jnp.concatenate(
        [batch, jnp.full((sc_pad,), G, jnp.int32)]) if sc_pad else batch

    roots = _roots_sc(x, batch_sc, n)
    return _pool_tc(x, batch3, roots, W1, b1r, W2, nblocks, blk)


# speedup vs baseline: 32.7640x; 1.2287x over previous
import functools

import jax
import jax.numpy as jnp
from jax import lax
from jax.experimental import pallas as pl
from jax.experimental.pallas import tpu as pltpu
from jax.experimental.pallas import tpu_sc as plsc

G = 256
D = 128
H = 32


def _roots_sc(x, batch_pad, n):
    NP = batch_pad.shape[0]
    NW = 16
    C = NP // NW
    T = 512
    mesh = plsc.VectorSubcoreMesh(core_axis_name="c", subcore_axis_name="s")

    @functools.partial(
        pl.kernel, mesh=mesh,
        out_type=jax.ShapeDtypeStruct((G, D), jnp.float32),
        scratch_types=[
            pltpu.VMEM((C + 16,), jnp.int32),
            pltpu.VMEM((T,), jnp.int32),
            pltpu.VMEM_SHARED((NW, T), jnp.int32),
            pltpu.VMEM((NW, T), jnp.int32),
            pltpu.VMEM((16, D), jnp.float32),
            pltpu.SemaphoreType.DMA,
        ],
        compiler_params=pltpu.CompilerParams(needs_layout_passes=False),
    )
    def k(batch_hbm, x_hbm, roots_hbm, buf, tbl, shared, tball, rows, sem):
        cid = lax.axis_index("c")
        sid = lax.axis_index("s")

        @pl.when(cid == 0)
        def _work():
            lo = sid * C
            zz = jnp.zeros((16,), jnp.int32)
            for t0 in range(0, T, 16):
                tbl[pl.ds(t0, 16)] = zz
            pltpu.sync_copy(batch_hbm.at[pl.ds(lo, C)], buf.at[pl.ds(16, C)])

            @pl.when(sid == 0)
            def _first():
                buf[pl.ds(0, 16)] = jnp.full((16,), -1, jnp.int32)

            @pl.when(sid > 0)
            def _rest():
                pltpu.sync_copy(batch_hbm.at[pl.ds(lo - 16, 16)],
                                buf.at[pl.ds(0, 16)])

            iota = lax.iota(jnp.int32, 16)

            def step(j, carry):
                cur = buf[pl.ds(16 + 16 * j, 16)]
                prv = buf[pl.ds(15 + 16 * j, 16)]
                bmask = cur != prv
                vals = lo + 16 * j + iota + 1
                plsc.store_scatter(tbl, [cur], vals, mask=bmask)
                return carry

            lax.fori_loop(0, C // 16, step, 0)

            pltpu.sync_copy(tbl, shared.at[sid])
            plsc.subcore_barrier()

            pltpu.sync_copy(shared, tball)
            acc = jnp.zeros((16,), jnp.int32)
            for t in range(NW):
                acc = acc + tball[t, pl.ds(16 * sid, 16)]
            fi = jnp.clip(acc - 1, 0, n - 1)
            pltpu.async_copy(x_hbm.at[fi], rows, sem).wait()
            pltpu.sync_copy(rows, roots_hbm.at[pl.ds(16 * sid, 16)])

    return k(batch_pad, x)


def _pool_body(nblocks, blk):
    def body(batch_ref, x_ref, roots_ref, W1_ref, b1_ref, W2_ref, out_ref,
             acc_ref, s_ref, rbc_ref):
        k = pl.program_id(0)
        bf = jnp.bfloat16

        @pl.when(k == 0)
        def _init():
            acc_ref[...] = jnp.zeros_like(acc_ref)
            s_ref[...] = jnp.zeros_like(s_ref)
            roots = roots_ref[...]
            W1a0 = W1_ref[0:D, :]
            W1b0 = W1_ref[D:2 * D, :]
            rbc = jnp.dot(roots, W1b0,
                          preferred_element_type=jnp.float32)
            pre_r = jnp.dot(roots, W1a0,
                            preferred_element_type=jnp.float32) \
                + rbc + b1_ref[...]
            beta_r = jnp.dot(jnp.tanh(pre_r), W2_ref[...],
                             preferred_element_type=jnp.float32)
            tab = jnp.concatenate([rbc, beta_r], axis=1)
            rbc_ref[...] = tab.astype(bf).T

        b = batch_ref[0, 0, :]
        xb = x_ref[...]
        W1a = W1_ref[0:D, :]

        gid = jax.lax.broadcasted_iota(jnp.int32, (G, 1), 0).astype(bf)
        b_bf = b[None, :].astype(bf)
        dd = gid - b_bf
        OfT = jnp.maximum(bf(1.0) - jnp.abs(dd), bf(0.0))

        xwaT = jnp.dot(xb, W1a,
                       preferred_element_type=jnp.float32).T

        gathT = jnp.dot(rbc_ref[...].astype(bf), OfT,
                        preferred_element_type=jnp.float32)
        hT = jnp.tanh(xwaT + gathT[0:H, :] + b1_ref[...].T)
        betaT = jax.lax.dot_general(
            W2_ref[...], hT, (((0,), (0,)), ((), ())),
            preferred_element_type=jnp.float32)

        eT = jnp.exp(betaT - gathT[H:H + 1, :])
        eOfT = OfT * eT.astype(bf)
        s_ref[...] = s_ref[...] + jnp.dot(
            eOfT, jnp.ones((blk, 1), bf),
            preferred_element_type=jnp.float32)
        acc_ref[...] = acc_ref[...] + jnp.dot(
            eOfT, xb.astype(bf),
            preferred_element_type=jnp.float32)

        @pl.when(k == nblocks - 1)
        def _finish():
            out_ref[...] = acc_ref[...] / (s_ref[...] + 1e-16)

    return body


def _pool_tc(x_pad, batch3, roots, W1, b1r, W2, nblocks, blk):
    return pl.pallas_call(
        _pool_body(nblocks, blk),
        grid=(nblocks,),
        in_specs=[
            pl.BlockSpec((1, 1, blk), lambda k: (k, 0, 0)),
            pl.BlockSpec((blk, D), lambda k: (k, 0)),
            pl.BlockSpec((G, D), lambda k: (0, 0)),
            pl.BlockSpec((2 * D, H), lambda k: (0, 0)),
            pl.BlockSpec((1, H), lambda k: (0, 0)),
            pl.BlockSpec((H, 1), lambda k: (0, 0)),
        ],
        out_specs=pl.BlockSpec((G, D), lambda k: (0, 0)),
        out_shape=jax.ShapeDtypeStruct((G, D), jnp.float32),
        scratch_shapes=[
            pltpu.VMEM((G, D), jnp.float32),
            pltpu.VMEM((G, 1), jnp.float32),
            pltpu.VMEM((H + 1, G), jnp.bfloat16),
        ],
        compiler_params=pltpu.CompilerParams(
            dimension_semantics=("arbitrary",)),
    )(batch3, x_pad, roots, W1, b1r, W2)


def kernel(x, batch, W1, b1, W2):
    n = x.shape[0]
    blk = 20000
    assert n % blk == 0
    nblocks = n // blk
    batch3 = batch.reshape(nblocks, 1, blk)
    b1r = b1.reshape(1, H)

    sc_pad = (-n) % (16 * 16)
    batch_sc = jnp.concatenate(
        [batch, jnp.full((sc_pad,), G, jnp.int32)]) if sc_pad else batch

    roots = _roots_sc(x, batch_sc, n)
    return _pool_tc(x, batch3, roots, W1, b1r, W2, nblocks, blk)
